# baseline (speedup 1.0000x reference)
import functools

import jax
import jax.numpy as jnp
from jax import lax
from jax.experimental import pallas as pl
from jax.experimental.pallas import tpu as pltpu
from jax.experimental.pallas import tpu_sc as plsc

B, C, K = 4, 96, 2048
H, W = 384, 384
HW = H * W
NW = 32
WPB = NW // B
RPW = H // WPB
PPW = RPW * W
CG = 8
NG = C // CG
RCH = 8
NCH = RPW // RCH
CHUNK = RCH * W
LANES = 16
NVEC = CHUNK // LANES
WVEC = W // LANES


def _unpool_body(pooled_hbm, spx_hbm, out_hbm,
                 idx_v, tab_v0, tab_v1, out_v0, out_v1,
                 sem_t, sem_o0, sem_o1):
    wid = lax.axis_index("s") * 2 + lax.axis_index("c")
    b = wid // WPB
    row0 = (wid % WPB) * RPW
    tabs = [tab_v0, tab_v1]
    outs = [out_v0, out_v1]
    osems = [sem_o0, sem_o1]

    pltpu.sync_copy(spx_hbm.at[pl.ds(b * HW + row0 * W, PPW)], idx_v)
    pltpu.sync_copy(pooled_hbm.at[pl.ds(b * C * K, CG * K)], tab_v0)

    def chunk_body(g, ch, tab, bi, first):
        out_v = outs[bi]
        dst = out_hbm.at[b, pl.ds(g * CG, CG), pl.ds(row0 + ch * RCH, RCH), :]
        if not first:
            pltpu.make_async_copy(out_v, dst, osems[bi]).wait()

        @plsc.parallel_loop(0, NVEC, unroll=12)
        def body(i):
            q = i // WVEC
            m = i % WVEC
            idx = idx_v[pl.ds(ch * CHUNK + i * LANES, LANES)]
            for c in range(CG):
                out_v[c, q, pl.ds(m * LANES, LANES)] = plsc.load_gather(
                    tab, [idx + c * K])

        pltpu.async_copy(out_v, dst, osems[bi])

    def tab_load(g, tv):
        return pltpu.async_copy(
            pooled_hbm.at[pl.ds((b * C + g * CG) * K, CG * K)], tv, sem_t)

    def tab_drain(tv):
        pltpu.make_async_copy(
            pooled_hbm.at[pl.ds(b * C * K, CG * K)], tv, sem_t).wait()

    def chunk_pairs(g, tab, lo):
        def pair(ch2, _):
            chunk_body(g, ch2 * 2, tab, 0, first=False)
            chunk_body(g, ch2 * 2 + 1, tab, 1, first=False)
            return 0

        lax.fori_loop(lo, NCH // 2, pair, 0)

    tab_load(1, tab_v1)
    chunk_body(0, 0, tab_v0, 0, first=True)
    chunk_body(0, 1, tab_v0, 1, first=True)
    chunk_pairs(0, tab_v0, 1)

    def gbody(gp, _):
        g1 = 2 * gp + 1
        tab_drain(tab_v1)
        tab_load(g1 + 1, tab_v0)
        chunk_pairs(g1, tab_v1, 0)
        tab_drain(tab_v0)
        tab_load(g1 + 2, tab_v1)
        chunk_pairs(g1 + 1, tab_v0, 0)
        return 0

    lax.fori_loop(0, (NG - 2) // 2, gbody, 0)

    tab_drain(tab_v1)
    chunk_pairs(NG - 1, tab_v1, 0)
    for bi in range(2):
        pltpu.make_async_copy(
            outs[bi],
            out_hbm.at[b, pl.ds(0, CG), pl.ds(row0, RCH), :],
            osems[bi]).wait()


@jax.jit
def kernel(pooled, spx):
    mesh = plsc.VectorSubcoreMesh(core_axis_name="c", subcore_axis_name="s")
    k = functools.partial(
        pl.kernel,
        out_type=jax.ShapeDtypeStruct((B, C, H, W), jnp.float32),
        mesh=mesh,
        compiler_params=pltpu.CompilerParams(needs_layout_passes=False),
        scratch_types=[
            pltpu.VMEM((PPW,), jnp.int32),
            pltpu.VMEM((CG * K,), jnp.float32),
            pltpu.VMEM((CG * K,), jnp.float32),
            pltpu.VMEM((CG, RCH, W), jnp.float32),
            pltpu.VMEM((CG, RCH, W), jnp.float32),
            pltpu.SemaphoreType.DMA,
            pltpu.SemaphoreType.DMA,
            pltpu.SemaphoreType.DMA,
        ],
    )(_unpool_body)
    return k(pooled.reshape(B * C * K), spx.reshape(B * HW))


# speedup vs baseline: 1.5426x; 1.5426x over previous
import functools

import jax
import jax.numpy as jnp
from jax import lax
from jax.experimental import pallas as pl
from jax.experimental.pallas import tpu as pltpu
from jax.experimental.pallas import tpu_sc as plsc

B, C, K = 4, 96, 2048
H, W = 384, 384
HW = H * W
NW = 32
WPB = NW // B
RPW = H // WPB
PPW = RPW * W
CG = 8
NG = C // CG
RCH = 8
NCH = RPW // RCH
CHUNK = RCH * W
LANES = 16
NVEC = CHUNK // LANES
WVEC = W // LANES


def _unpool_body(pooled_hbm, spx_hbm, out_hbm,
                 idx_v, tab_v0, tab_v1, out_v0, out_v1,
                 sem_t, sem_o0, sem_o1):
    wid = lax.axis_index("s") * 2 + lax.axis_index("c")
    b = wid // WPB
    row0 = (wid % WPB) * RPW
    tabs = [tab_v0, tab_v1]
    outs = [out_v0, out_v1]
    osems = [sem_o0, sem_o1]

    pltpu.sync_copy(spx_hbm.at[pl.ds(b * HW + row0 * W, PPW)], idx_v)
    pltpu.sync_copy(pooled_hbm.at[pl.ds(b * C * K, CG * K)], tab_v0)

    def chunk_body(g, ch, tab, bi, first):
        out_v = outs[bi]
        dst = out_hbm.at[b, pl.ds(g * CG, CG), pl.ds(row0 + ch * RCH, RCH), :]
        if not first:
            pltpu.make_async_copy(out_v, dst, osems[bi]).wait()

        @plsc.parallel_loop(0, NVEC, unroll=4)
        def body(i):
            q = i // WVEC
            m = i % WVEC
            idx = idx_v[pl.ds(ch * CHUNK + i * LANES, LANES)]
            for c in range(CG):
                out_v[c, q, pl.ds(m * LANES, LANES)] = plsc.load_gather(
                    tab, [idx + c * K])

        pltpu.async_copy(out_v, dst, osems[bi])

    def tab_load(g, tv):
        return pltpu.async_copy(
            pooled_hbm.at[pl.ds((b * C + g * CG) * K, CG * K)], tv, sem_t)

    def tab_drain(tv):
        pltpu.make_async_copy(
            pooled_hbm.at[pl.ds(b * C * K, CG * K)], tv, sem_t).wait()

    def chunk_pairs(g, tab, lo):
        def pair(ch2, _):
            chunk_body(g, ch2 * 2, tab, 0, first=False)
            chunk_body(g, ch2 * 2 + 1, tab, 1, first=False)
            return 0

        lax.fori_loop(lo, NCH // 2, pair, 0)

    tab_load(1, tab_v1)
    chunk_body(0, 0, tab_v0, 0, first=True)
    chunk_body(0, 1, tab_v0, 1, first=True)
    chunk_pairs(0, tab_v0, 1)

    def gbody(gp, _):
        g1 = 2 * gp + 1
        tab_drain(tab_v1)
        tab_load(g1 + 1, tab_v0)
        chunk_pairs(g1, tab_v1, 0)
        tab_drain(tab_v0)
        tab_load(g1 + 2, tab_v1)
        chunk_pairs(g1 + 1, tab_v0, 0)
        return 0

    lax.fori_loop(0, (NG - 2) // 2, gbody, 0)

    tab_drain(tab_v1)
    chunk_pairs(NG - 1, tab_v1, 0)
    for bi in range(2):
        pltpu.make_async_copy(
            outs[bi],
            out_hbm.at[b, pl.ds(0, CG), pl.ds(row0, RCH), :],
            osems[bi]).wait()


@jax.jit
def kernel(pooled, spx):
    mesh = plsc.VectorSubcoreMesh(core_axis_name="c", subcore_axis_name="s")
    k = functools.partial(
        pl.kernel,
        out_type=jax.ShapeDtypeStruct((B, C, H, W), jnp.float32),
        mesh=mesh,
        compiler_params=pltpu.CompilerParams(needs_layout_passes=False),
        scratch_types=[
            pltpu.VMEM((PPW,), jnp.int32),
            pltpu.VMEM((CG * K,), jnp.float32),
            pltpu.VMEM((CG * K,), jnp.float32),
            pltpu.VMEM((CG, RCH, W), jnp.float32),
            pltpu.VMEM((CG, RCH, W), jnp.float32),
            pltpu.SemaphoreType.DMA,
            pltpu.SemaphoreType.DMA,
            pltpu.SemaphoreType.DMA,
        ],
    )(_unpool_body)
    return k(pooled.reshape(B * C * K), spx.reshape(B * HW))


# speedup vs baseline: 1.8966x; 1.2295x over previous
import functools

import jax
import jax.numpy as jnp
from jax import lax
from jax.experimental import pallas as pl
from jax.experimental.pallas import tpu as pltpu
from jax.experimental.pallas import tpu_sc as plsc

B, C, K = 4, 96, 2048
H, W = 384, 384
HW = H * W
NW = 32
WPB = NW // B
RPW = H // WPB
PPW = RPW * W
CG = 8
NG = C // CG
RCH = 8
NCH = RPW // RCH
CHUNK = RCH * W
LANES = 16
NVEC = CHUNK // LANES
WVEC = W // LANES


def _unpool_body(pooled_hbm, spx_hbm, out_hbm,
                 idx_v, tab_v0, tab_v1, out_v0, out_v1,
                 sem_t, sem_o0, sem_o1):
    wid = lax.axis_index("s") * 2 + lax.axis_index("c")
    b = wid // WPB
    row0 = (wid % WPB) * RPW
    tabs = [tab_v0, tab_v1]
    outs = [out_v0, out_v1]
    osems = [sem_o0, sem_o1]

    pltpu.sync_copy(spx_hbm.at[pl.ds(b * HW + row0 * W, PPW)], idx_v)
    pltpu.sync_copy(pooled_hbm.at[pl.ds(b * C * K, CG * K)], tab_v0)

    def chunk_body(g, ch, tab, bi, first):
        out_v = outs[bi]
        dst = out_hbm.at[b, pl.ds(g * CG, CG), pl.ds(row0 + ch * RCH, RCH), :]
        if not first:
            pltpu.make_async_copy(out_v, dst, osems[bi]).wait()

        @plsc.parallel_loop(0, NVEC, unroll=8)
        def body(i):
            q = i // WVEC
            m = i % WVEC
            idx = idx_v[pl.ds(ch * CHUNK + i * LANES, LANES)]
            for c in range(CG):
                out_v[c, q, pl.ds(m * LANES, LANES)] = plsc.load_gather(
                    tab, [idx + c * K])

        pltpu.async_copy(out_v, dst, osems[bi])

    def tab_load(g, tv):
        return pltpu.async_copy(
            pooled_hbm.at[pl.ds((b * C + g * CG) * K, CG * K)], tv, sem_t)

    def tab_drain(tv):
        pltpu.make_async_copy(
            pooled_hbm.at[pl.ds(b * C * K, CG * K)], tv, sem_t).wait()

    def chunk_pairs(g, tab, lo):
        def pair(ch2, _):
            chunk_body(g, ch2 * 2, tab, 0, first=False)
            chunk_body(g, ch2 * 2 + 1, tab, 1, first=False)
            return 0

        lax.fori_loop(lo, NCH // 2, pair, 0)

    tab_load(1, tab_v1)
    chunk_body(0, 0, tab_v0, 0, first=True)
    chunk_body(0, 1, tab_v0, 1, first=True)
    chunk_pairs(0, tab_v0, 1)

    def gbody(gp, _):
        g1 = 2 * gp + 1
        tab_drain(tab_v1)
        tab_load(g1 + 1, tab_v0)
        chunk_pairs(g1, tab_v1, 0)
        tab_drain(tab_v0)
        tab_load(g1 + 2, tab_v1)
        chunk_pairs(g1 + 1, tab_v0, 0)
        return 0

    lax.fori_loop(0, (NG - 2) // 2, gbody, 0)

    tab_drain(tab_v1)
    chunk_pairs(NG - 1, tab_v1, 0)
    for bi in range(2):
        pltpu.make_async_copy(
            outs[bi],
            out_hbm.at[b, pl.ds(0, CG), pl.ds(row0, RCH), :],
            osems[bi]).wait()


@jax.jit
def kernel(pooled, spx):
    mesh = plsc.VectorSubcoreMesh(core_axis_name="c", subcore_axis_name="s")
    k = functools.partial(
        pl.kernel,
        out_type=jax.ShapeDtypeStruct((B, C, H, W), jnp.float32),
        mesh=mesh,
        compiler_params=pltpu.CompilerParams(needs_layout_passes=False),
        scratch_types=[
            pltpu.VMEM((PPW,), jnp.int32),
            pltpu.VMEM((CG * K,), jnp.float32),
            pltpu.VMEM((CG * K,), jnp.float32),
            pltpu.VMEM((CG, RCH, W), jnp.float32),
            pltpu.VMEM((CG, RCH, W), jnp.float32),
            pltpu.SemaphoreType.DMA,
            pltpu.SemaphoreType.DMA,
            pltpu.SemaphoreType.DMA,
        ],
    )(_unpool_body)
    return k(pooled.reshape(B * C * K), spx.reshape(B * HW))


# speedup vs baseline: 2.0511x; 1.0814x over previous
import functools

import jax
import jax.numpy as jnp
from jax import lax
from jax.experimental import pallas as pl
from jax.experimental.pallas import tpu as pltpu
from jax.experimental.pallas import tpu_sc as plsc

B, C, K = 4, 96, 2048
H, W = 384, 384
HW = H * W
NW = 32
WPB = NW // B
RPW = H // WPB
PPW = RPW * W
CG = 8
NG = C // CG
RCH = 8
NCH = RPW // RCH
CHUNK = RCH * W
LANES = 16
NVEC = CHUNK // LANES
WVEC = W // LANES


def _unpool_body(pooled_hbm, spx_hbm, out_hbm,
                 idx_v, tab_v0, tab_v1, out_v0, out_v1,
                 sem_t, sem_o0, sem_o1):
    wid = lax.axis_index("s") * 2 + lax.axis_index("c")
    b = wid // WPB
    row0 = (wid % WPB) * RPW
    tabs = [tab_v0, tab_v1]
    outs = [out_v0, out_v1]
    osems = [sem_o0, sem_o1]

    pltpu.sync_copy(spx_hbm.at[pl.ds(b * HW + row0 * W, PPW)], idx_v)
    pltpu.sync_copy(pooled_hbm.at[pl.ds(b * C * K, CG * K)], tab_v0)

    def chunk_body(g, ch, tab, bi, first):
        out_v = outs[bi]
        dst = out_hbm.at[b, pl.ds(g * CG, CG), pl.ds(row0 + ch * RCH, RCH), :]
        if not first:
            pltpu.make_async_copy(out_v, dst, osems[bi]).wait()

        @plsc.parallel_loop(0, NVEC, unroll=3)
        def body(i):
            q = i // WVEC
            m = i % WVEC
            idx = idx_v[pl.ds(ch * CHUNK + i * LANES, LANES)]
            for c in range(CG):
                out_v[c, q, pl.ds(m * LANES, LANES)] = plsc.load_gather(
                    tab, [idx + c * K])

        pltpu.async_copy(out_v, dst, osems[bi])

    for g in range(NG):
        tab_pend = None
        if g + 1 < NG:
            tab_pend = pltpu.async_copy(
                pooled_hbm.at[pl.ds((b * C + (g + 1) * CG) * K, CG * K)],
                tabs[(g + 1) % 2], sem_t)
        tab = tabs[g % 2]
        chunk_body(g, 0, tab, 0, first=(g == 0))
        chunk_body(g, 1, tab, 1, first=(g == 0))

        def pair(ch2, _):
            chunk_body(g, ch2 * 2, tab, 0, first=False)
            chunk_body(g, ch2 * 2 + 1, tab, 1, first=False)
            return 0

        lax.fori_loop(1, NCH // 2, pair, 0)
        if tab_pend is not None:
            tab_pend.wait()
    for bi in range(2):
        pltpu.make_async_copy(
            outs[bi],
            out_hbm.at[b, pl.ds(0, CG), pl.ds(row0, RCH), :],
            osems[bi]).wait()


@jax.jit
def kernel(pooled, spx):
    mesh = plsc.VectorSubcoreMesh(core_axis_name="c", subcore_axis_name="s")
    k = functools.partial(
        pl.kernel,
        out_type=jax.ShapeDtypeStruct((B, C, H, W), jnp.float32),
        mesh=mesh,
        compiler_params=pltpu.CompilerParams(needs_layout_passes=False),
        scratch_types=[
            pltpu.VMEM((PPW,), jnp.int32),
            pltpu.VMEM((CG * K,), jnp.float32),
            pltpu.VMEM((CG * K,), jnp.float32),
            pltpu.VMEM((CG, RCH, W), jnp.float32),
            pltpu.VMEM((CG, RCH, W), jnp.float32),
            pltpu.SemaphoreType.DMA,
            pltpu.SemaphoreType.DMA,
            pltpu.SemaphoreType.DMA,
        ],
    )(_unpool_body)
    return k(pooled.reshape(B * C * K), spx.reshape(B * HW))
